# baseline (speedup 1.0000x reference)
import jax
import jax.numpy as jnp
from jax.experimental import pallas as pl
from jax.experimental.pallas import tpu as pltpu

F32 = jnp.float32

_B, _NQ, _WAY, _SHOT = 2, 15, 5, 5
_H1, _W1P = 84, 88
_P1 = _H1 * _W1P
_H2, _W2P = 42, 48
_P2 = _H2 * _W2P
_H3, _W3P = 21, 25
_P3 = _H3 * _W3P
_HW = 21 * 21
_D = 64
_NIMG = _B * _NQ + _B * _WAY * _SHOT
_GIMG = 4
_GQ = 1
_M = _SHOT * _HW

_LPAD = 128
_RPAD2, _RPAD3 = 56, 32

_OFF1 = [di * _W1P + dj for di in (-1, 0, 1) for dj in (-1, 0, 1)]
_OFF2 = [di * _W2P + dj for di in (-1, 0, 1) for dj in (-1, 0, 1)]
_OFF3 = [di * _W3P + dj for di in (-1, 0, 1) for dj in (-1, 0, 1)]


def _leaky(x):
    return jnp.where(x >= 0, x, 0.2 * x)


def _conv9(src_ref, w_ref, li, offs, rpad, P):
    acc = None
    for t, off in enumerate(offs):
        xs = src_ref[rpad + off: rpad + off + P, :]
        d = jnp.dot(xs, w_ref[li, t], preferred_element_type=F32)
        acc = d if acc is None else acc + d
    return acc


def _enc_body_q(x_ref, w1_ref, w_ref, b_ref, o_ref, s1, s2, s3, s4, t1, t2):
    s1[...] = jnp.zeros(s1.shape, F32)
    s2[...] = jnp.zeros(s2.shape, F32)
    s3[...] = jnp.zeros(s3.shape, F32)
    s4[...] = jnp.zeros(s4.shape, F32)
    for g in range(_GIMG):
        o_ref[g] = _enc_one(x_ref, w1_ref, w_ref, b_ref, s1, s2, s3, s4, t1, t2, g)


def _enc_body_s(x_ref, w1_ref, w_ref, b_ref, o_ref, s1, s2, s3, s4, t1, t2):
    s1[...] = jnp.zeros(s1.shape, F32)
    s2[...] = jnp.zeros(s2.shape, F32)
    s3[...] = jnp.zeros(s3.shape, F32)
    s4[...] = jnp.zeros(s4.shape, F32)
    for g in range(_SHOT):
        yn = _enc_one(x_ref, w1_ref, w_ref, b_ref, s1, s2, s3, s4, t1, t2, g)
        o_ref[0, :, g * _HW: (g + 1) * _HW] = yn


def _enc_one(x_ref, w1_ref, w_ref, b_ref, s1, s2, s3, s4, t1, t2, g):
    s1[0:3, _LPAD: _LPAD + _P1] = x_ref[g]
    xk = jnp.concatenate(
        [s1[:, _LPAD + off: _LPAD + off + _P1] for off in _OFF1], axis=0)
    y = jax.lax.dot_general(xk, w1_ref[...], (((0,), (0,)), ((), ())),
                            preferred_element_type=F32)

    t1[...] = y
    a = jnp.maximum(t1[0::2, :], t1[1::2, :])
    half1 = _W1P // 2
    for i2 in range(_H2):
        r0 = (2 * i2) * half1
        blk = jnp.maximum(a[r0: r0 + half1, :], a[r0 + half1: r0 + 2 * half1, :])
        base = _RPAD2 + i2 * _W2P
        s2[base + 2: base + 2 + _H2, :] = _leaky(blk[1: 1 + _H2, :] + b_ref[0][None, :])

    y = _conv9(s2, w_ref, 0, _OFF2, _RPAD2, _P2)
    t2[...] = y
    a = jnp.maximum(t2[0::2, :], t2[1::2, :])
    half2 = _W2P // 2
    for i2 in range(_H3):
        r0 = (2 * i2) * half2
        blk = jnp.maximum(a[r0: r0 + half2, :], a[r0 + half2: r0 + 2 * half2, :])
        base = _RPAD3 + i2 * _W3P
        s3[base + 2: base + 2 + _H3, :] = _leaky(blk[1: 1 + _H3, :] + b_ref[1][None, :])

    y = _leaky(_conv9(s3, w_ref, 1, _OFF3, _RPAD3, _P3) + b_ref[2][None, :])
    for i in range(_H3):
        s4[_RPAD3 + i * _W3P + 2: _RPAD3 + i * _W3P + 2 + _H3, :] = \
            y[i * _W3P + 2: i * _W3P + 2 + _H3, :]

    y = _leaky(_conv9(s4, w_ref, 2, _OFF3, _RPAD3, _P3) + b_ref[3][None, :])
    yc = jnp.concatenate(
        [y[i * _W3P + 2: i * _W3P + 2 + _H3, :] for i in range(_H3)], axis=0)
    yt = yc.T
    n = jnp.sqrt(jnp.sum(yt * yt, axis=0, keepdims=True))
    return yt / jnp.clip(n, 1e-12)


def _score_body(q_ref, s_ref, o_ref):
    for qg in range(_GQ):
        q = q_ref[qg]
        for c in range(_WAY):
            s = s_ref[0, c * _D: (c + 1) * _D, :]
            sim = jax.lax.dot_general(q, s, (((0,), (0,)), ((), ())),
                                      preferred_element_type=F32)
            neg = jnp.float32(-jnp.inf)
            m1 = jnp.max(sim, axis=1, keepdims=True)
            e1 = sim == m1
            ones = jnp.ones((_M, 1), F32)
            c1 = jnp.dot(e1.astype(F32), ones, preferred_element_type=F32)
            sim2 = jnp.where(e1, neg, sim)
            m2 = jnp.max(sim2, axis=1, keepdims=True)
            e2 = sim2 == m2
            c2 = jnp.dot(e2.astype(F32), ones, preferred_element_type=F32)
            sim3 = jnp.where(e2, neg, sim2)
            m3 = jnp.max(sim3, axis=1, keepdims=True)
            second = jnp.where(c1 >= 2, m1, m2)
            third = jnp.where(c1 >= 3, m1, jnp.where(c1 + c2 >= 3, m2, m3))
            o_ref[qg, c] = jnp.sum(m1 + second + third, axis=0, keepdims=True)


def kernel(query, support, W1, b1, g1, t1, W2, b2, g2, t2, W3, b3, g3, t3, W4, b4, g4, t4):
    def prep(W, b, g, t):
        Wf = W * g[:, None, None, None]
        taps = jnp.transpose(Wf, (2, 3, 1, 0))
        taps = taps.reshape(9, W.shape[1], 64)
        return taps, b * g + t

    w1p, bias1 = prep(W1, b1, g1, t1)
    w1k = jnp.pad(w1p, ((0, 0), (0, 5), (0, 0))).reshape(72, 64)
    w2p, bias2 = prep(W2, b2, g2, t2)
    w3p, bias3 = prep(W3, b3, g3, t3)
    w4p, bias4 = prep(W4, b4, g4, t4)
    wp = jnp.stack([w2p, w3p, w4p])
    biases = jnp.stack([bias1, bias2, bias3, bias4])

    q_imgs = query.reshape(-1, 3, _H1, _H1)
    s_imgs = support.reshape(-1, 3, _H1, _H1)
    imgs = jnp.concatenate([q_imgs, s_imgs], 0)
    imgs = jnp.pad(imgs, ((0, 0), (0, 0), (0, 0), (2, 2))).reshape(_NIMG, 3, _P1)

    scratches = [
        pltpu.VMEM((8, _P1 + 2 * _LPAD), F32),
        pltpu.VMEM((_P2 + 2 * _RPAD2, 64), F32),
        pltpu.VMEM((_P3 + 2 * _RPAD3, 64), F32),
        pltpu.VMEM((_P3 + 2 * _RPAD3, 64), F32),
        pltpu.VMEM((_P1, 64), F32),
        pltpu.VMEM((_P2, 64), F32),
    ]
    w_specs = [
        pl.BlockSpec((72, 64), lambda i: (0, 0)),
        pl.BlockSpec((3, 9, 64, 64), lambda i: (0, 0, 0, 0)),
        pl.BlockSpec((4, 64), lambda i: (0, 0)),
    ]
    nq_total = _B * _NQ

    feats = pl.pallas_call(
        _enc_body_q,
        grid=(_NIMG // _GIMG,),
        in_specs=[pl.BlockSpec((_GIMG, 3, _P1), lambda i: (i, 0, 0))] + w_specs,
        out_specs=pl.BlockSpec((_GIMG, _D, _HW), lambda i: (i, 0, 0)),
        out_shape=jax.ShapeDtypeStruct((_NIMG, _D, _HW), F32),
        scratch_shapes=scratches,
        compiler_params=pltpu.CompilerParams(
            dimension_semantics=("parallel",)),
    )(imgs, w1k, wp, biases)

    qn = feats[: nq_total]
    sn = feats[nq_total:].reshape(_B, _WAY, _SHOT, _D, _HW)
    st = jnp.transpose(sn, (0, 1, 3, 2, 4)).reshape(_B, _WAY * _D, _M)

    scores = pl.pallas_call(
        _score_body,
        grid=(nq_total // _GQ,),
        in_specs=[
            pl.BlockSpec((_GQ, _D, _HW), lambda qi: (qi, 0, 0)),
            pl.BlockSpec((1, _WAY * _D, _M), lambda qi: (qi * _GQ // _NQ, 0, 0)),
        ],
        out_specs=pl.BlockSpec((_GQ, _WAY, 1, 1), lambda qi: (qi, 0, 0, 0)),
        out_shape=jax.ShapeDtypeStruct((nq_total, _WAY, 1, 1), F32),
        compiler_params=pltpu.CompilerParams(
            dimension_semantics=("parallel",)),
    )(qn, st)

    return scores.reshape(nq_total, _WAY)


# speedup vs baseline: 1.0711x; 1.0711x over previous
import jax
import jax.numpy as jnp
from jax.experimental import pallas as pl
from jax.experimental.pallas import tpu as pltpu

F32 = jnp.float32

_B, _NQ, _WAY, _SHOT = 2, 15, 5, 5
_H1, _W1P = 84, 88
_P1 = _H1 * _W1P
_H2, _W2P = 42, 48
_P2 = _H2 * _W2P
_H3, _W3P = 21, 25
_P3 = _H3 * _W3P
_HW = 21 * 21
_D = 64
_NIMG = _B * _NQ + _B * _WAY * _SHOT
_GIMG = 4
_GQ = 1
_M = _SHOT * _HW

_LPAD = 128
_RPAD2, _RPAD3 = 56, 32

_OFF1 = [di * _W1P + dj for di in (-1, 0, 1) for dj in (-1, 0, 1)]
_OFF2 = [di * _W2P + dj for di in (-1, 0, 1) for dj in (-1, 0, 1)]
_OFF3 = [di * _W3P + dj for di in (-1, 0, 1) for dj in (-1, 0, 1)]


def _leaky(x):
    return jnp.where(x >= 0, x, 0.2 * x)


def _conv9(src_ref, w_ref, li, offs, rpad, P):
    acc = None
    for t, off in enumerate(offs):
        xs = src_ref[rpad + off: rpad + off + P, :]
        d = jnp.dot(xs, w_ref[li, t], preferred_element_type=F32)
        acc = d if acc is None else acc + d
    return acc


def _enc_body_q(x_ref, w1_ref, w_ref, b_ref, o_ref, s1, s2, s3, s4, t1, t2):
    s1[...] = jnp.zeros(s1.shape, F32)
    s2[...] = jnp.zeros(s2.shape, F32)
    s3[...] = jnp.zeros(s3.shape, F32)
    s4[...] = jnp.zeros(s4.shape, F32)
    for g in range(_GIMG):
        o_ref[g] = _enc_one(x_ref, w1_ref, w_ref, b_ref, s1, s2, s3, s4, t1, t2, g)


def _enc_body_s(x_ref, w1_ref, w_ref, b_ref, o_ref, s1, s2, s3, s4, t1, t2):
    s1[...] = jnp.zeros(s1.shape, F32)
    s2[...] = jnp.zeros(s2.shape, F32)
    s3[...] = jnp.zeros(s3.shape, F32)
    s4[...] = jnp.zeros(s4.shape, F32)
    for g in range(_SHOT):
        yn = _enc_one(x_ref, w1_ref, w_ref, b_ref, s1, s2, s3, s4, t1, t2, g)
        o_ref[0, :, g * _HW: (g + 1) * _HW] = yn


def _enc_one(x_ref, w1_ref, w_ref, b_ref, s1, s2, s3, s4, t1, t2, g):
    s1[0:3, _LPAD: _LPAD + _P1] = x_ref[g]
    xk = jnp.concatenate(
        [s1[:, _LPAD + off: _LPAD + off + _P1] for off in _OFF1], axis=0)
    y = jax.lax.dot_general(xk, w1_ref[...], (((0,), (0,)), ((), ())),
                            preferred_element_type=F32)

    t1[...] = y
    a = jnp.maximum(t1[0::2, :], t1[1::2, :])
    half1 = _W1P // 2
    for i2 in range(_H2):
        r0 = (2 * i2) * half1
        blk = jnp.maximum(a[r0: r0 + half1, :], a[r0 + half1: r0 + 2 * half1, :])
        base = _RPAD2 + i2 * _W2P
        s2[base + 2: base + 2 + _H2, :] = _leaky(blk[1: 1 + _H2, :] + b_ref[0][None, :])

    y = _conv9(s2, w_ref, 0, _OFF2, _RPAD2, _P2)
    t2[...] = y
    a = jnp.maximum(t2[0::2, :], t2[1::2, :])
    half2 = _W2P // 2
    for i2 in range(_H3):
        r0 = (2 * i2) * half2
        blk = jnp.maximum(a[r0: r0 + half2, :], a[r0 + half2: r0 + 2 * half2, :])
        base = _RPAD3 + i2 * _W3P
        s3[base + 2: base + 2 + _H3, :] = _leaky(blk[1: 1 + _H3, :] + b_ref[1][None, :])

    y = _leaky(_conv9(s3, w_ref, 1, _OFF3, _RPAD3, _P3) + b_ref[2][None, :])
    for i in range(_H3):
        s4[_RPAD3 + i * _W3P + 2: _RPAD3 + i * _W3P + 2 + _H3, :] = \
            y[i * _W3P + 2: i * _W3P + 2 + _H3, :]

    y = _leaky(_conv9(s4, w_ref, 2, _OFF3, _RPAD3, _P3) + b_ref[3][None, :])
    yc = jnp.concatenate(
        [y[i * _W3P + 2: i * _W3P + 2 + _H3, :] for i in range(_H3)], axis=0)
    yt = yc.T
    n = jnp.sqrt(jnp.sum(yt * yt, axis=0, keepdims=True))
    return yt / jnp.clip(n, 1e-12)


def _score_body(q_ref, s_ref, o_ref):
    for qg in range(_GQ):
        q = q_ref[qg]
        for c in range(_WAY):
            s = s_ref[0, c * _D: (c + 1) * _D, :]
            sim = jax.lax.dot_general(q, s, (((0,), (0,)), ((), ())),
                                      preferred_element_type=F32)
            neg = jnp.float32(-jnp.inf)
            m1 = jnp.max(sim, axis=1, keepdims=True)
            e1 = sim == m1
            c1 = jnp.sum(e1.astype(F32), axis=1, keepdims=True)
            sim2 = jnp.where(e1, neg, sim)
            m2 = jnp.max(sim2, axis=1, keepdims=True)
            e2 = sim2 == m2
            c2 = jnp.sum(e2.astype(F32), axis=1, keepdims=True)
            sim3 = jnp.where(e2, neg, sim2)
            m3 = jnp.max(sim3, axis=1, keepdims=True)
            second = jnp.where(c1 >= 2, m1, m2)
            third = jnp.where(c1 >= 3, m1, jnp.where(c1 + c2 >= 3, m2, m3))
            o_ref[qg, c] = jnp.sum(m1 + second + third, axis=0, keepdims=True)


def kernel(query, support, W1, b1, g1, t1, W2, b2, g2, t2, W3, b3, g3, t3, W4, b4, g4, t4):
    def prep(W, b, g, t):
        Wf = W * g[:, None, None, None]
        taps = jnp.transpose(Wf, (2, 3, 1, 0))
        taps = taps.reshape(9, W.shape[1], 64)
        return taps, b * g + t

    w1p, bias1 = prep(W1, b1, g1, t1)
    w1k = jnp.pad(w1p, ((0, 0), (0, 5), (0, 0))).reshape(72, 64)
    w2p, bias2 = prep(W2, b2, g2, t2)
    w3p, bias3 = prep(W3, b3, g3, t3)
    w4p, bias4 = prep(W4, b4, g4, t4)
    wp = jnp.stack([w2p, w3p, w4p])
    biases = jnp.stack([bias1, bias2, bias3, bias4])

    q_imgs = query.reshape(-1, 3, _H1, _H1)
    s_imgs = support.reshape(-1, 3, _H1, _H1)
    imgs = jnp.concatenate([q_imgs, s_imgs], 0)
    imgs = jnp.pad(imgs, ((0, 0), (0, 0), (0, 0), (2, 2))).reshape(_NIMG, 3, _P1)

    scratches = [
        pltpu.VMEM((8, _P1 + 2 * _LPAD), F32),
        pltpu.VMEM((_P2 + 2 * _RPAD2, 64), F32),
        pltpu.VMEM((_P3 + 2 * _RPAD3, 64), F32),
        pltpu.VMEM((_P3 + 2 * _RPAD3, 64), F32),
        pltpu.VMEM((_P1, 64), F32),
        pltpu.VMEM((_P2, 64), F32),
    ]
    w_specs = [
        pl.BlockSpec((72, 64), lambda i: (0, 0)),
        pl.BlockSpec((3, 9, 64, 64), lambda i: (0, 0, 0, 0)),
        pl.BlockSpec((4, 64), lambda i: (0, 0)),
    ]
    nq_total = _B * _NQ

    feats = pl.pallas_call(
        _enc_body_q,
        grid=(_NIMG // _GIMG,),
        in_specs=[pl.BlockSpec((_GIMG, 3, _P1), lambda i: (i, 0, 0))] + w_specs,
        out_specs=pl.BlockSpec((_GIMG, _D, _HW), lambda i: (i, 0, 0)),
        out_shape=jax.ShapeDtypeStruct((_NIMG, _D, _HW), F32),
        scratch_shapes=scratches,
        compiler_params=pltpu.CompilerParams(
            dimension_semantics=("parallel",)),
    )(imgs, w1k, wp, biases)

    qn = feats[: nq_total]
    sn = feats[nq_total:].reshape(_B, _WAY, _SHOT, _D, _HW)
    st = jnp.transpose(sn, (0, 1, 3, 2, 4)).reshape(_B, _WAY * _D, _M)

    scores = pl.pallas_call(
        _score_body,
        grid=(nq_total // _GQ,),
        in_specs=[
            pl.BlockSpec((_GQ, _D, _HW), lambda qi: (qi, 0, 0)),
            pl.BlockSpec((1, _WAY * _D, _M), lambda qi: (qi * _GQ // _NQ, 0, 0)),
        ],
        out_specs=pl.BlockSpec((_GQ, _WAY, 1, 1), lambda qi: (qi, 0, 0, 0)),
        out_shape=jax.ShapeDtypeStruct((nq_total, _WAY, 1, 1), F32),
        compiler_params=pltpu.CompilerParams(
            dimension_semantics=("parallel",)),
    )(qn, st)

    return scores.reshape(nq_total, _WAY)


# speedup vs baseline: 1.2120x; 1.1316x over previous
import jax
import jax.numpy as jnp
from jax.experimental import pallas as pl
from jax.experimental.pallas import tpu as pltpu

F32 = jnp.float32

_B, _NQ, _WAY, _SHOT = 2, 15, 5, 5
_H1, _W1P = 84, 88
_P1 = _H1 * _W1P
_H2, _W2P = 42, 48
_P2 = _H2 * _W2P
_H3, _W3P = 21, 25
_P3 = _H3 * _W3P
_HW = 21 * 21
_D = 64
_NIMG = _B * _NQ + _B * _WAY * _SHOT
_GIMG = 4
_GQ = 1
_M = _SHOT * _HW

_LPAD = 128
_RPAD2, _RPAD3 = 56, 32

_OFF1 = [di * _W1P + dj for di in (-1, 0, 1) for dj in (-1, 0, 1)]
_OFF2 = [di * _W2P + dj for di in (-1, 0, 1) for dj in (-1, 0, 1)]
_OFF3 = [di * _W3P + dj for di in (-1, 0, 1) for dj in (-1, 0, 1)]


def _leaky(x):
    return jnp.where(x >= 0, x, 0.2 * x)


def _conv9(src_ref, w_ref, li, offs, rpad, P):
    acc = None
    for t, off in enumerate(offs):
        xs = src_ref[rpad + off: rpad + off + P, :]
        d = jnp.dot(xs, w_ref[li, t], preferred_element_type=F32)
        acc = d if acc is None else acc + d
    return acc


def _enc_body_q(x_ref, w1_ref, w_ref, b_ref, nm_ref, o_ref, s1, s2, s3, s4, t1, t2):
    s1[...] = jnp.zeros(s1.shape, F32)
    s2[...] = jnp.zeros(s2.shape, F32)
    s3[...] = jnp.zeros(s3.shape, F32)
    s4[...] = jnp.zeros(s4.shape, F32)
    for g in range(_GIMG // 2):
        ynt = _enc_pair(x_ref, w1_ref, w_ref, b_ref, nm_ref,
                        s1, s2, s3, s4, t1, t2, g)
        o_ref[2 * g] = ynt[0:_D, :]
        o_ref[2 * g + 1] = ynt[_D:2 * _D, :]


def _enc_pair(x_ref, w1_ref, w_ref, b_ref, nm_ref, s1, s2, s3, s4, t1, t2, g):
    s1[0:3, _LPAD: _LPAD + _P1] = x_ref[2 * g]
    s1[8:11, _LPAD: _LPAD + _P1] = x_ref[2 * g + 1]
    xk = jnp.concatenate(
        [s1[:, _LPAD + off: _LPAD + off + _P1] for off in _OFF1], axis=0)
    y = jax.lax.dot_general(xk, w1_ref[...], (((0,), (0,)), ((), ())),
                            preferred_element_type=F32)

    t1[...] = y
    a = jnp.maximum(t1[0::2, :], t1[1::2, :])
    half1 = _W1P // 2
    for i2 in range(_H2):
        r0 = (2 * i2) * half1
        blk = jnp.maximum(a[r0: r0 + half1, :], a[r0 + half1: r0 + 2 * half1, :])
        base = _RPAD2 + i2 * _W2P
        s2[base + 2: base + 2 + _H2, :] = _leaky(blk[1: 1 + _H2, :] + b_ref[0][None, :])

    y = _conv9(s2, w_ref, 0, _OFF2, _RPAD2, _P2)
    t2[...] = y
    a = jnp.maximum(t2[0::2, :], t2[1::2, :])
    half2 = _W2P // 2
    for i2 in range(_H3):
        r0 = (2 * i2) * half2
        blk = jnp.maximum(a[r0: r0 + half2, :], a[r0 + half2: r0 + 2 * half2, :])
        base = _RPAD3 + i2 * _W3P
        s3[base + 2: base + 2 + _H3, :] = _leaky(blk[1: 1 + _H3, :] + b_ref[1][None, :])

    y = _leaky(_conv9(s3, w_ref, 1, _OFF3, _RPAD3, _P3) + b_ref[2][None, :])
    for i in range(_H3):
        s4[_RPAD3 + i * _W3P + 2: _RPAD3 + i * _W3P + 2 + _H3, :] = \
            y[i * _W3P + 2: i * _W3P + 2 + _H3, :]

    y = _leaky(_conv9(s4, w_ref, 2, _OFF3, _RPAD3, _P3) + b_ref[3][None, :])
    yc = jnp.concatenate(
        [y[i * _W3P + 2: i * _W3P + 2 + _H3, :] for i in range(_H3)], axis=0)
    nsq = jnp.dot(yc * yc, nm_ref[...], preferred_element_type=F32)
    inv = 1.0 / jnp.clip(jnp.sqrt(nsq), 1e-12)
    mul = jnp.concatenate(
        [jnp.broadcast_to(inv[:, 0:1], (_HW, _D)),
         jnp.broadcast_to(inv[:, 1:2], (_HW, _D))], axis=1)
    return (yc * mul).T


def _score_body(q_ref, s_ref, o_ref):
    for qg in range(_GQ):
        q = q_ref[qg]
        for c in range(_WAY):
            s = s_ref[0, c * _D: (c + 1) * _D, :]
            sim = jax.lax.dot_general(q, s, (((0,), (0,)), ((), ())),
                                      preferred_element_type=F32)
            neg = jnp.float32(-jnp.inf)
            m1 = jnp.max(sim, axis=1, keepdims=True)
            e1 = sim == m1
            c1 = jnp.sum(e1.astype(F32), axis=1, keepdims=True)
            sim2 = jnp.where(e1, neg, sim)
            m2 = jnp.max(sim2, axis=1, keepdims=True)
            e2 = sim2 == m2
            c2 = jnp.sum(e2.astype(F32), axis=1, keepdims=True)
            sim3 = jnp.where(e2, neg, sim2)
            m3 = jnp.max(sim3, axis=1, keepdims=True)
            second = jnp.where(c1 >= 2, m1, m2)
            third = jnp.where(c1 >= 3, m1, jnp.where(c1 + c2 >= 3, m2, m3))
            o_ref[qg, c] = jnp.sum(m1 + second + third, axis=0, keepdims=True)


def kernel(query, support, W1, b1, g1, t1, W2, b2, g2, t2, W3, b3, g3, t3, W4, b4, g4, t4):
    def prep(W, b, g, t):
        Wf = W * g[:, None, None, None]
        taps = jnp.transpose(Wf, (2, 3, 1, 0))
        taps = taps.reshape(9, W.shape[1], 64)
        return taps, b * g + t

    w1p, bias1 = prep(W1, b1, g1, t1)
    w1a = jnp.pad(w1p, ((0, 0), (0, 5), (0, 0)))
    w1k = jnp.concatenate(
        [jnp.pad(w1a, ((0, 0), (0, 0), (0, 64))),
         jnp.pad(w1a, ((0, 0), (0, 0), (64, 0)))], axis=1).reshape(144, 128)
    w2p, bias2 = prep(W2, b2, g2, t2)
    w3p, bias3 = prep(W3, b3, g3, t3)
    w4p, bias4 = prep(W4, b4, g4, t4)
    wps = jnp.stack([w2p, w3p, w4p])
    z = jnp.zeros_like(wps)
    wp = jnp.concatenate([jnp.concatenate([wps, z], -1),
                          jnp.concatenate([z, wps], -1)], -2)
    biases = jnp.stack([bias1, bias2, bias3, bias4])
    biases = jnp.concatenate([biases, biases], axis=1)
    eye = jnp.ones((_D, 1), F32)
    zv = jnp.zeros((_D, 1), F32)
    nmask = jnp.concatenate([jnp.concatenate([eye, zv], 1),
                             jnp.concatenate([zv, eye], 1)], 0)

    q_imgs = query.reshape(-1, 3, _H1, _H1)
    s_imgs = support.reshape(-1, 3, _H1, _H1)
    imgs = jnp.concatenate([q_imgs, s_imgs], 0)
    imgs = jnp.pad(imgs, ((0, 0), (0, 0), (0, 0), (2, 2))).reshape(_NIMG, 3, _P1)

    scratches = [
        pltpu.VMEM((16, _P1 + 2 * _LPAD), F32),
        pltpu.VMEM((_P2 + 2 * _RPAD2, 128), F32),
        pltpu.VMEM((_P3 + 2 * _RPAD3, 128), F32),
        pltpu.VMEM((_P3 + 2 * _RPAD3, 128), F32),
        pltpu.VMEM((_P1, 128), F32),
        pltpu.VMEM((_P2, 128), F32),
    ]
    w_specs = [
        pl.BlockSpec((144, 128), lambda i: (0, 0)),
        pl.BlockSpec((3, 9, 128, 128), lambda i: (0, 0, 0, 0)),
        pl.BlockSpec((4, 128), lambda i: (0, 0)),
        pl.BlockSpec((128, 2), lambda i: (0, 0)),
    ]
    nq_total = _B * _NQ

    feats = pl.pallas_call(
        _enc_body_q,
        grid=(_NIMG // _GIMG,),
        in_specs=[pl.BlockSpec((_GIMG, 3, _P1), lambda i: (i, 0, 0))] + w_specs,
        out_specs=pl.BlockSpec((_GIMG, _D, _HW), lambda i: (i, 0, 0)),
        out_shape=jax.ShapeDtypeStruct((_NIMG, _D, _HW), F32),
        scratch_shapes=scratches,
        compiler_params=pltpu.CompilerParams(
            dimension_semantics=("parallel",)),
    )(imgs, w1k, wp, biases, nmask)

    qn = feats[: nq_total]
    sn = feats[nq_total:].reshape(_B, _WAY, _SHOT, _D, _HW)
    st = jnp.transpose(sn, (0, 1, 3, 2, 4)).reshape(_B, _WAY * _D, _M)

    scores = pl.pallas_call(
        _score_body,
        grid=(nq_total // _GQ,),
        in_specs=[
            pl.BlockSpec((_GQ, _D, _HW), lambda qi: (qi, 0, 0)),
            pl.BlockSpec((1, _WAY * _D, _M), lambda qi: (qi * _GQ // _NQ, 0, 0)),
        ],
        out_specs=pl.BlockSpec((_GQ, _WAY, 1, 1), lambda qi: (qi, 0, 0, 0)),
        out_shape=jax.ShapeDtypeStruct((nq_total, _WAY, 1, 1), F32),
        compiler_params=pltpu.CompilerParams(
            dimension_semantics=("parallel",)),
    )(qn, st)

    return scores.reshape(nq_total, _WAY)


# speedup vs baseline: 1.2481x; 1.0298x over previous
import jax
import jax.numpy as jnp
from jax.experimental import pallas as pl
from jax.experimental.pallas import tpu as pltpu

F32 = jnp.float32

_B, _NQ, _WAY, _SHOT = 2, 15, 5, 5
_H1, _W1P = 84, 88
_P1 = _H1 * _W1P
_H2, _W2P = 42, 48
_P2 = _H2 * _W2P
_H3, _W3P = 21, 25
_P3 = _H3 * _W3P
_HW = 21 * 21
_D = 64
_NIMG = _B * _NQ + _B * _WAY * _SHOT
_GIMG = 8
_GQ = 1
_M = _SHOT * _HW

_LPAD = 128
_RPAD2, _RPAD3 = 56, 32

_OFF1 = [di * _W1P + dj for di in (-1, 0, 1) for dj in (-1, 0, 1)]
_OFF2 = [di * _W2P + dj for di in (-1, 0, 1) for dj in (-1, 0, 1)]
_OFF3 = [di * _W3P + dj for di in (-1, 0, 1) for dj in (-1, 0, 1)]


def _leaky(x):
    return jnp.where(x >= 0, x, 0.2 * x)


def _conv9(src_ref, w_ref, li, offs, rpad, P):
    acc = None
    for t, off in enumerate(offs):
        xs = src_ref[rpad + off: rpad + off + P, :]
        d = jnp.dot(xs, w_ref[li, t], preferred_element_type=F32)
        acc = d if acc is None else acc + d
    return acc


def _enc_body_q(x_ref, w1_ref, w_ref, b_ref, nm_ref, o_ref, s1, s2, s3, s4, t1, t2):
    s1[...] = jnp.zeros(s1.shape, F32)
    s2[...] = jnp.zeros(s2.shape, F32)
    s3[...] = jnp.zeros(s3.shape, F32)
    s4[...] = jnp.zeros(s4.shape, F32)
    for g in range(_GIMG // 2):
        ynt = _enc_pair(x_ref, w1_ref, w_ref, b_ref, nm_ref,
                        s1, s2, s3, s4, t1, t2, g)
        o_ref[2 * g] = ynt[0:_D, :]
        o_ref[2 * g + 1] = ynt[_D:2 * _D, :]


def _enc_pair(x_ref, w1_ref, w_ref, b_ref, nm_ref, s1, s2, s3, s4, t1, t2, g):
    s1[0:3, _LPAD: _LPAD + _P1] = x_ref[2 * g]
    s1[8:11, _LPAD: _LPAD + _P1] = x_ref[2 * g + 1]
    xk = jnp.concatenate(
        [s1[:, _LPAD + off: _LPAD + off + _P1] for off in _OFF1], axis=0)
    y = jax.lax.dot_general(xk, w1_ref[...], (((0,), (0,)), ((), ())),
                            preferred_element_type=F32)

    t1[...] = y
    a = jnp.maximum(t1[0::2, :], t1[1::2, :])
    half1 = _W1P // 2
    for i2 in range(_H2):
        r0 = (2 * i2) * half1
        blk = jnp.maximum(a[r0: r0 + half1, :], a[r0 + half1: r0 + 2 * half1, :])
        base = _RPAD2 + i2 * _W2P
        s2[base + 2: base + 2 + _H2, :] = _leaky(blk[1: 1 + _H2, :] + b_ref[0][None, :])

    y = _conv9(s2, w_ref, 0, _OFF2, _RPAD2, _P2)
    t2[...] = y
    a = jnp.maximum(t2[0::2, :], t2[1::2, :])
    half2 = _W2P // 2
    for i2 in range(_H3):
        r0 = (2 * i2) * half2
        blk = jnp.maximum(a[r0: r0 + half2, :], a[r0 + half2: r0 + 2 * half2, :])
        base = _RPAD3 + i2 * _W3P
        s3[base + 2: base + 2 + _H3, :] = _leaky(blk[1: 1 + _H3, :] + b_ref[1][None, :])

    y = _leaky(_conv9(s3, w_ref, 1, _OFF3, _RPAD3, _P3) + b_ref[2][None, :])
    for i in range(_H3):
        s4[_RPAD3 + i * _W3P + 2: _RPAD3 + i * _W3P + 2 + _H3, :] = \
            y[i * _W3P + 2: i * _W3P + 2 + _H3, :]

    y = _leaky(_conv9(s4, w_ref, 2, _OFF3, _RPAD3, _P3) + b_ref[3][None, :])
    yc = jnp.concatenate(
        [y[i * _W3P + 2: i * _W3P + 2 + _H3, :] for i in range(_H3)], axis=0)
    nsq = jnp.dot(yc * yc, nm_ref[...], preferred_element_type=F32)
    inv = 1.0 / jnp.clip(jnp.sqrt(nsq), 1e-12)
    mul = jnp.concatenate(
        [jnp.broadcast_to(inv[:, 0:1], (_HW, _D)),
         jnp.broadcast_to(inv[:, 1:2], (_HW, _D))], axis=1)
    return (yc * mul).T


def _score_body(q_ref, s_ref, o_ref):
    for qg in range(_GQ):
        q = q_ref[qg]
        for c in range(_WAY):
            s = s_ref[0, c * _D: (c + 1) * _D, :]
            sim = jax.lax.dot_general(q, s, (((0,), (0,)), ((), ())),
                                      preferred_element_type=F32)
            neg = jnp.float32(-jnp.inf)
            m1 = jnp.max(sim, axis=1, keepdims=True)
            e1 = sim == m1
            c1 = jnp.sum(e1.astype(F32), axis=1, keepdims=True)
            sim2 = jnp.where(e1, neg, sim)
            m2 = jnp.max(sim2, axis=1, keepdims=True)
            e2 = sim2 == m2
            c2 = jnp.sum(e2.astype(F32), axis=1, keepdims=True)
            sim3 = jnp.where(e2, neg, sim2)
            m3 = jnp.max(sim3, axis=1, keepdims=True)
            second = jnp.where(c1 >= 2, m1, m2)
            third = jnp.where(c1 >= 3, m1, jnp.where(c1 + c2 >= 3, m2, m3))
            o_ref[qg, c] = jnp.sum(m1 + second + third, axis=0, keepdims=True)


def kernel(query, support, W1, b1, g1, t1, W2, b2, g2, t2, W3, b3, g3, t3, W4, b4, g4, t4):
    def prep(W, b, g, t):
        Wf = W * g[:, None, None, None]
        taps = jnp.transpose(Wf, (2, 3, 1, 0))
        taps = taps.reshape(9, W.shape[1], 64)
        return taps, b * g + t

    w1p, bias1 = prep(W1, b1, g1, t1)
    w1a = jnp.pad(w1p, ((0, 0), (0, 5), (0, 0)))
    w1k = jnp.concatenate(
        [jnp.pad(w1a, ((0, 0), (0, 0), (0, 64))),
         jnp.pad(w1a, ((0, 0), (0, 0), (64, 0)))], axis=1).reshape(144, 128)
    w2p, bias2 = prep(W2, b2, g2, t2)
    w3p, bias3 = prep(W3, b3, g3, t3)
    w4p, bias4 = prep(W4, b4, g4, t4)
    wps = jnp.stack([w2p, w3p, w4p])
    z = jnp.zeros_like(wps)
    wp = jnp.concatenate([jnp.concatenate([wps, z], -1),
                          jnp.concatenate([z, wps], -1)], -2)
    biases = jnp.stack([bias1, bias2, bias3, bias4])
    biases = jnp.concatenate([biases, biases], axis=1)
    eye = jnp.ones((_D, 1), F32)
    zv = jnp.zeros((_D, 1), F32)
    nmask = jnp.concatenate([jnp.concatenate([eye, zv], 1),
                             jnp.concatenate([zv, eye], 1)], 0)

    q_imgs = query.reshape(-1, 3, _H1, _H1)
    s_imgs = support.reshape(-1, 3, _H1, _H1)
    imgs = jnp.concatenate([q_imgs, s_imgs], 0)
    imgs = jnp.pad(imgs, ((0, 0), (0, 0), (0, 0), (2, 2))).reshape(_NIMG, 3, _P1)

    scratches = [
        pltpu.VMEM((16, _P1 + 2 * _LPAD), F32),
        pltpu.VMEM((_P2 + 2 * _RPAD2, 128), F32),
        pltpu.VMEM((_P3 + 2 * _RPAD3, 128), F32),
        pltpu.VMEM((_P3 + 2 * _RPAD3, 128), F32),
        pltpu.VMEM((_P1, 128), F32),
        pltpu.VMEM((_P2, 128), F32),
    ]
    w_specs = [
        pl.BlockSpec((144, 128), lambda i: (0, 0)),
        pl.BlockSpec((3, 9, 128, 128), lambda i: (0, 0, 0, 0)),
        pl.BlockSpec((4, 128), lambda i: (0, 0)),
        pl.BlockSpec((128, 2), lambda i: (0, 0)),
    ]
    nq_total = _B * _NQ

    feats = pl.pallas_call(
        _enc_body_q,
        grid=(_NIMG // _GIMG,),
        in_specs=[pl.BlockSpec((_GIMG, 3, _P1), lambda i: (i, 0, 0))] + w_specs,
        out_specs=pl.BlockSpec((_GIMG, _D, _HW), lambda i: (i, 0, 0)),
        out_shape=jax.ShapeDtypeStruct((_NIMG, _D, _HW), F32),
        scratch_shapes=scratches,
        compiler_params=pltpu.CompilerParams(
            dimension_semantics=("parallel",)),
    )(imgs, w1k, wp, biases, nmask)

    qn = feats[: nq_total]
    sn = feats[nq_total:].reshape(_B, _WAY, _SHOT, _D, _HW)
    st = jnp.transpose(sn, (0, 1, 3, 2, 4)).reshape(_B, _WAY * _D, _M)

    scores = pl.pallas_call(
        _score_body,
        grid=(nq_total // _GQ,),
        in_specs=[
            pl.BlockSpec((_GQ, _D, _HW), lambda qi: (qi, 0, 0)),
            pl.BlockSpec((1, _WAY * _D, _M), lambda qi: (qi * _GQ // _NQ, 0, 0)),
        ],
        out_specs=pl.BlockSpec((_GQ, _WAY, 1, 1), lambda qi: (qi, 0, 0, 0)),
        out_shape=jax.ShapeDtypeStruct((nq_total, _WAY, 1, 1), F32),
        compiler_params=pltpu.CompilerParams(
            dimension_semantics=("parallel",)),
    )(qn, st)

    return scores.reshape(nq_total, _WAY)
